# baseline (speedup 1.0000x reference)
import functools

import jax
import jax.numpy as jnp
from jax import lax
from jax.experimental import pallas as pl
from jax.experimental.pallas import tpu as pltpu
from jax.experimental.pallas import tpu_sc as plsc

C = 4096
NUM_PEDESTALS = 16
PED_FEATS = 4
SP_FEATS = 2
OUT_FEATS = PED_FEATS + SP_FEATS

_info = plsc.get_sparse_core_info()
NS, L = _info.num_subcores, _info.num_lanes
CPW = C // NS
ROWS_PER_VREG = L // PED_FEATS
STEPS = CPW // ROWS_PER_VREG


def _sc_body(idx_hbm, sp_hbm, tbl_hbm, out_hbm, idx_v, tbl_v, sp_v, out_v,
             sem):
    wid = lax.axis_index("s")
    base = wid * CPW

    cp_idx = pltpu.async_copy(idx_hbm.at[pl.ds(base, CPW)], idx_v, sem)
    cp_tbl = pltpu.async_copy(tbl_hbm, tbl_v, sem)
    cp_sp = pltpu.async_copy(sp_hbm.at[pl.ds(base, CPW), :], sp_v, sem)
    cp_idx.wait()
    cp_tbl.wait()

    lanes = lax.iota(jnp.int32, L)
    sub_row = lanes // PED_FEATS
    col = lanes - sub_row * PED_FEATS
    for t in range(STEPS):
        row = sub_row + t * ROWS_PER_VREG
        ped = plsc.load_gather(idx_v, [row])
        val = plsc.load_gather(tbl_v, [ped, col])
        plsc.store_scatter(out_v, [row, col], val)

    cp_sp.wait()
    sp_row = lanes // SP_FEATS
    sp_col = lanes - sp_row * SP_FEATS
    for t in range(CPW * SP_FEATS // L):
        row = sp_row + t * (L // SP_FEATS)
        val = plsc.load_gather(sp_v, [row, sp_col])
        plsc.store_scatter(out_v, [row, PED_FEATS + sp_col], val)

    pltpu.sync_copy(out_v, out_hbm.at[pl.ds(base, CPW), :])


_sc_call = functools.partial(
    pl.kernel,
    mesh=plsc.VectorSubcoreMesh(core_axis_name="c", subcore_axis_name="s",
                                num_cores=1),
    out_type=jax.ShapeDtypeStruct((C, OUT_FEATS), jnp.float32),
    scratch_types=[
        pltpu.VMEM((CPW,), jnp.int32),
        pltpu.VMEM((NUM_PEDESTALS, PED_FEATS), jnp.float32),
        pltpu.VMEM((CPW, SP_FEATS), jnp.float32),
        pltpu.VMEM((CPW, OUT_FEATS), jnp.float32),
        pltpu.SemaphoreType.DMA,
    ],
    compiler_params=pltpu.CompilerParams(
        needs_layout_passes=False,
        disable_bounds_checks=True,
        skip_device_barrier=True,
    ),
)(_sc_body)


@jax.jit
def kernel(pedestals, spatial_embeddings, pedestal_table):
    return _sc_call(pedestals.astype(jnp.int32), spatial_embeddings,
                    pedestal_table)


# speedup vs baseline: 1.0631x; 1.0631x over previous
import functools

import jax
import jax.numpy as jnp
from jax import lax
from jax.experimental import pallas as pl
from jax.experimental.pallas import tpu as pltpu
from jax.experimental.pallas import tpu_sc as plsc

C = 4096
NUM_PEDESTALS = 16
PED_FEATS = 4
SP_FEATS = 2
OUT_FEATS = PED_FEATS + SP_FEATS

_info = plsc.get_sparse_core_info()
NS, L = _info.num_subcores, _info.num_lanes
CPW = C // NS
ROWS_PER_VREG = L // PED_FEATS
STEPS = CPW // ROWS_PER_VREG


def _sc_body(idx_hbm, sp_hbm, tbl_hbm, out_hbm, idx_v, tbl_v, sp_v, out_v,
             sem):
    wid = lax.axis_index("s")
    base = wid * CPW

    cp_idx = pltpu.async_copy(idx_hbm.at[pl.ds(base, CPW)], idx_v, sem)
    cp_tbl = pltpu.async_copy(tbl_hbm, tbl_v, sem)
    cp_sp = pltpu.async_copy(sp_hbm.at[pl.ds(base, CPW), :], sp_v, sem)
    cp_idx.wait()
    cp_tbl.wait()

    lanes = lax.iota(jnp.int32, L)
    sub_row = lanes // PED_FEATS
    col = lanes - sub_row * PED_FEATS

    UNROLL = 8

    def tbl_step(t, carry):
        for k in range(UNROLL):
            row = sub_row + (t * UNROLL + k) * ROWS_PER_VREG
            ped = plsc.load_gather(idx_v, [row])
            val = plsc.load_gather(tbl_v, [ped, col])
            plsc.store_scatter(out_v, [row, col], val)
        return carry

    lax.fori_loop(0, STEPS // UNROLL, tbl_step, 0)

    cp_sp.wait()
    sp_row = lanes // SP_FEATS
    sp_col = lanes - sp_row * SP_FEATS
    SP_STEPS = CPW * SP_FEATS // L

    def sp_step(t, carry):
        for k in range(UNROLL):
            row = sp_row + (t * UNROLL + k) * (L // SP_FEATS)
            val = plsc.load_gather(sp_v, [row, sp_col])
            plsc.store_scatter(out_v, [row, PED_FEATS + sp_col], val)
        return carry

    lax.fori_loop(0, SP_STEPS // UNROLL, sp_step, 0)

    pltpu.sync_copy(out_v, out_hbm.at[pl.ds(base, CPW), :])


_sc_call = functools.partial(
    pl.kernel,
    mesh=plsc.VectorSubcoreMesh(core_axis_name="c", subcore_axis_name="s",
                                num_cores=1),
    out_type=jax.ShapeDtypeStruct((C, OUT_FEATS), jnp.float32),
    scratch_types=[
        pltpu.VMEM((CPW,), jnp.int32),
        pltpu.VMEM((NUM_PEDESTALS, PED_FEATS), jnp.float32),
        pltpu.VMEM((CPW, SP_FEATS), jnp.float32),
        pltpu.VMEM((CPW, OUT_FEATS), jnp.float32),
        pltpu.SemaphoreType.DMA,
    ],
    compiler_params=pltpu.CompilerParams(
        needs_layout_passes=False,
        disable_bounds_checks=True,
        skip_device_barrier=True,
    ),
)(_sc_body)


@jax.jit
def kernel(pedestals, spatial_embeddings, pedestal_table):
    return _sc_call(pedestals.astype(jnp.int32), spatial_embeddings,
                    pedestal_table)


# speedup vs baseline: 1.1401x; 1.0724x over previous
import functools

import jax
import jax.numpy as jnp
from jax import lax
from jax.experimental import pallas as pl
from jax.experimental.pallas import tpu as pltpu
from jax.experimental.pallas import tpu_sc as plsc

C = 4096
NUM_PEDESTALS = 16
PED_FEATS = 4
SP_FEATS = 2
OUT_FEATS = PED_FEATS + SP_FEATS

_info = plsc.get_sparse_core_info()
NS, L = _info.num_subcores, _info.num_lanes
CPW = C // NS
ROWS_PER_VREG = L // PED_FEATS
STEPS = CPW // ROWS_PER_VREG


def _sc_body(idx_hbm, sp_hbm, tbl_hbm, out_hbm, idx_v, tbl_v, sp_v, out_v,
             sem):
    wid = lax.axis_index("s")
    base = wid * CPW

    cp_idx = pltpu.async_copy(idx_hbm.at[pl.ds(base, CPW)], idx_v, sem)
    cp_tbl = pltpu.async_copy(tbl_hbm, tbl_v, sem)
    cp_sp = pltpu.async_copy(sp_hbm.at[pl.ds(base, CPW), :], sp_v, sem)
    cp_idx.wait()
    cp_tbl.wait()

    lanes = lax.iota(jnp.int32, L)
    sub_row = lanes // PED_FEATS
    col = lanes - sub_row * PED_FEATS

    UNROLL = 8

    def tbl_step(t, carry):
        for k in range(UNROLL):
            row = sub_row + (t * UNROLL + k) * ROWS_PER_VREG
            ped = plsc.load_gather(idx_v, [row])
            val = plsc.load_gather(tbl_v, [ped, col])
            plsc.store_scatter(out_v, [row, col], val)
        return carry

    lax.fori_loop(0, STEPS // UNROLL, tbl_step, 0)

    cp_sp.wait()
    sp_row = lanes // SP_FEATS
    sp_col = lanes - sp_row * SP_FEATS
    SP_STEPS = CPW * SP_FEATS // L

    def sp_step(t, carry):
        for k in range(UNROLL):
            row = sp_row + (t * UNROLL + k) * (L // SP_FEATS)
            val = plsc.load_gather(sp_v, [row, sp_col])
            plsc.store_scatter(out_v, [row, PED_FEATS + sp_col], val)
        return carry

    lax.fori_loop(0, SP_STEPS // UNROLL, sp_step, 0)

    pltpu.sync_copy(out_v, out_hbm.at[pl.ds(base, CPW), :])


_sc_call = functools.partial(
    pl.kernel,
    mesh=plsc.VectorSubcoreMesh(core_axis_name="c", subcore_axis_name="s",
                                num_cores=1),
    out_type=jax.ShapeDtypeStruct((C, OUT_FEATS), jnp.float32),
    scratch_types=[
        pltpu.VMEM((CPW,), jnp.int32),
        pltpu.VMEM((NUM_PEDESTALS, PED_FEATS), jnp.float32),
        pltpu.VMEM((CPW, SP_FEATS), jnp.float32),
        pltpu.VMEM((CPW, OUT_FEATS), jnp.float32),
        pltpu.SemaphoreType.DMA,
    ],
    compiler_params=pltpu.CompilerParams(
        needs_layout_passes=False,
        disable_bounds_checks=True,
        skip_device_barrier=True,
        use_tc_tiling_on_sc=False,
    ),
)(_sc_body)


@jax.jit
def kernel(pedestals, spatial_embeddings, pedestal_table):
    return _sc_call(pedestals.astype(jnp.int32), spatial_embeddings,
                    pedestal_table)


# speedup vs baseline: 1.3115x; 1.1504x over previous
import functools

import jax
import jax.numpy as jnp
from jax import lax
from jax.experimental import pallas as pl
from jax.experimental.pallas import tpu as pltpu
from jax.experimental.pallas import tpu_sc as plsc

C = 4096
NUM_PEDESTALS = 16
PED_FEATS = 4
SP_FEATS = 2
OUT_FEATS = PED_FEATS + SP_FEATS

_info = plsc.get_sparse_core_info()
NS, L = _info.num_subcores, _info.num_lanes
CPW = C // NS
ROWS_PER_VREG = L // PED_FEATS
STEPS = CPW // ROWS_PER_VREG
UNROLL = 8


def _sc_body(idx_hbm, sp_hbm, tbl_hbm, out_hbm, idx_v, tbl_v, sp_v, out_v,
             sem):
    wid = lax.axis_index("s")
    base = wid * CPW

    cp_idx = pltpu.async_copy(idx_hbm.at[pl.ds(base, CPW)], idx_v, sem)
    cp_tbl = pltpu.async_copy(tbl_hbm, tbl_v, sem)
    cp_sp = pltpu.async_copy(sp_hbm.at[:, pl.ds(base, CPW)], sp_v, sem)
    cp_idx.wait()
    cp_tbl.wait()

    lanes = lax.iota(jnp.int32, L)
    sub_row = lanes // PED_FEATS
    col = lanes - sub_row * PED_FEATS

    def tbl_step(t, carry):
        for k in range(UNROLL):
            row = sub_row + (t * UNROLL + k) * ROWS_PER_VREG
            ped = plsc.load_gather(idx_v, [row])
            val = plsc.load_gather(tbl_v, [col, ped])
            plsc.store_scatter(out_v, [col, row], val)
        return carry

    lax.fori_loop(0, STEPS // UNROLL, tbl_step, 0)

    cp_sp.wait()
    sp_row = lanes // SP_FEATS
    sp_col = lanes - sp_row * SP_FEATS
    SP_STEPS = CPW * SP_FEATS // L

    def sp_step(t, carry):
        for k in range(UNROLL):
            row = sp_row + (t * UNROLL + k) * (L // SP_FEATS)
            val = plsc.load_gather(sp_v, [sp_col, row])
            plsc.store_scatter(out_v, [PED_FEATS + sp_col, row], val)
        return carry

    lax.fori_loop(0, SP_STEPS // UNROLL, sp_step, 0)

    pltpu.sync_copy(out_v, out_hbm.at[:, pl.ds(base, CPW)])


_sc_call = functools.partial(
    pl.kernel,
    mesh=plsc.VectorSubcoreMesh(core_axis_name="c", subcore_axis_name="s",
                                num_cores=1),
    out_type=jax.ShapeDtypeStruct((OUT_FEATS, C), jnp.float32),
    scratch_types=[
        pltpu.VMEM((CPW,), jnp.int32),
        pltpu.VMEM((PED_FEATS, NUM_PEDESTALS), jnp.float32),
        pltpu.VMEM((SP_FEATS, CPW), jnp.float32),
        pltpu.VMEM((OUT_FEATS, CPW), jnp.float32),
        pltpu.SemaphoreType.DMA,
    ],
    compiler_params=pltpu.CompilerParams(
        needs_layout_passes=False,
        disable_bounds_checks=True,
        skip_device_barrier=True,
        use_tc_tiling_on_sc=False,
    ),
)(_sc_body)


@jax.jit
def kernel(pedestals, spatial_embeddings, pedestal_table):
    out_t = _sc_call(pedestals.astype(jnp.int32), spatial_embeddings.T,
                     pedestal_table.T)
    return out_t.T
